# Initial kernel scaffold; baseline (speedup 1.0000x reference)
import jax
import jax.numpy as jnp
from jax.experimental import pallas as pl


def kernel(input_features, unpool_map):
    raise NotImplementedError("write your pallas kernel here")


# speedup vs baseline: 2.5717x; 2.5717x over previous
import functools

import jax
import jax.numpy as jnp
from jax import lax
from jax.experimental import pallas as pl
from jax.experimental.pallas import tpu as pltpu
from jax.experimental.pallas import tpu_sc as plsc

N_IN_ROWS = 50000
N_OUT_ROWS = 400000
FEAT = 128

NUM_CORES = 2
NUM_SUBCORES = 16
NUM_WORKERS = NUM_CORES * NUM_SUBCORES

CHUNK = 128
CHUNKS_PER_WORKER = 98
ROWS_PER_WORKER = CHUNK * CHUNKS_PER_WORKER
PADDED_ROWS = ROWS_PER_WORKER * NUM_WORKERS


def _gather_body(table_hbm, idx_hbm, out_hbm, idx_v, rows_v, idx_sem, row_sem):
    wid = lax.axis_index("s") * NUM_CORES + lax.axis_index("c")
    base = wid * ROWS_PER_WORKER

    pltpu.async_copy(idx_hbm.at[pl.ds(base, ROWS_PER_WORKER)], idx_v, idx_sem).wait()

    def chunk_step(c, _):
        pltpu.async_copy(
            table_hbm.at[idx_v.at[pl.ds(c * CHUNK, CHUNK)]], rows_v, row_sem
        ).wait()
        pltpu.async_copy(
            rows_v, out_hbm.at[pl.ds(base + c * CHUNK, CHUNK)], row_sem
        ).wait()
        return 0

    lax.fori_loop(0, CHUNKS_PER_WORKER, chunk_step, 0)


@jax.jit
def _unpool_gather(table, idx):
    mesh = plsc.VectorSubcoreMesh(core_axis_name="c", subcore_axis_name="s")
    run = functools.partial(
        pl.kernel,
        mesh=mesh,
        out_type=jax.ShapeDtypeStruct((PADDED_ROWS, FEAT), jnp.float32),
        scratch_types=[
            pltpu.VMEM((ROWS_PER_WORKER,), jnp.int32),
            pltpu.VMEM((CHUNK, FEAT), jnp.float32),
            pltpu.SemaphoreType.DMA,
            pltpu.SemaphoreType.DMA,
        ],
    )(_gather_body)
    return run(table, idx)


def kernel(input_features, unpool_map):
    idx = unpool_map.astype(jnp.int32)
    pad = PADDED_ROWS - N_OUT_ROWS
    pad_idx = jnp.arange(pad, dtype=jnp.int32) % N_IN_ROWS
    idx_full = jnp.concatenate([idx, pad_idx])
    out = _unpool_gather(input_features, idx_full)
    return out[:N_OUT_ROWS]


# speedup vs baseline: 2.8418x; 1.1050x over previous
import functools

import jax
import jax.numpy as jnp
from jax import lax
from jax.experimental import pallas as pl
from jax.experimental.pallas import tpu as pltpu
from jax.experimental.pallas import tpu_sc as plsc

N_IN_ROWS = 50000
N_OUT_ROWS = 400000
FEAT = 128

NUM_CORES = 2
NUM_SUBCORES = 16
NUM_WORKERS = NUM_CORES * NUM_SUBCORES

CHUNK = 128
CHUNKS_PER_WORKER = 98
ROWS_PER_WORKER = CHUNK * CHUNKS_PER_WORKER
PADDED_ROWS = ROWS_PER_WORKER * NUM_WORKERS


def _gather_body(
    table_hbm, idx_hbm, out_hbm, idx_v, rows0, rows1, idx_sem, gat_sem, wb0_sem, wb1_sem
):
    wid = lax.axis_index("s") * NUM_CORES + lax.axis_index("c")
    base = wid * ROWS_PER_WORKER

    pltpu.async_copy(idx_hbm.at[pl.ds(base, ROWS_PER_WORKER)], idx_v, idx_sem).wait()

    bufs = (rows0, rows1)
    wb_sems = (wb0_sem, wb1_sem)

    def chunk_pair(g, _):
        for b in range(2):
            c = 2 * g + b

            @pl.when(c >= 2)
            def _wait_prev_wb():
                pltpu.make_async_copy(
                    bufs[b], out_hbm.at[pl.ds(0, CHUNK)], wb_sems[b]
                ).wait()

            pltpu.async_copy(
                table_hbm.at[idx_v.at[pl.ds(c * CHUNK, CHUNK)]], bufs[b], gat_sem
            ).wait()
            pltpu.async_copy(
                bufs[b], out_hbm.at[pl.ds(base + c * CHUNK, CHUNK)], wb_sems[b]
            )
        return 0

    lax.fori_loop(0, CHUNKS_PER_WORKER // 2, chunk_pair, 0)
    pltpu.make_async_copy(rows0, out_hbm.at[pl.ds(0, CHUNK)], wb0_sem).wait()
    pltpu.make_async_copy(rows1, out_hbm.at[pl.ds(0, CHUNK)], wb1_sem).wait()


@jax.jit
def _unpool_gather(table, idx):
    mesh = plsc.VectorSubcoreMesh(core_axis_name="c", subcore_axis_name="s")
    run = functools.partial(
        pl.kernel,
        mesh=mesh,
        out_type=jax.ShapeDtypeStruct((PADDED_ROWS, FEAT), jnp.float32),
        scratch_types=[
            pltpu.VMEM((ROWS_PER_WORKER,), jnp.int32),
            pltpu.VMEM((CHUNK, FEAT), jnp.float32),
            pltpu.VMEM((CHUNK, FEAT), jnp.float32),
            pltpu.SemaphoreType.DMA,
            pltpu.SemaphoreType.DMA,
            pltpu.SemaphoreType.DMA,
            pltpu.SemaphoreType.DMA,
        ],
    )(_gather_body)
    return run(table, idx)


def kernel(input_features, unpool_map):
    idx = unpool_map.astype(jnp.int32)
    pad = PADDED_ROWS - N_OUT_ROWS
    pad_idx = jnp.arange(pad, dtype=jnp.int32) % N_IN_ROWS
    idx_full = jnp.concatenate([idx, pad_idx])
    out = _unpool_gather(input_features, idx_full)
    return out[:N_OUT_ROWS]


# speedup vs baseline: 3.1925x; 1.1234x over previous
import functools

import jax
import jax.numpy as jnp
from jax import lax
from jax.experimental import pallas as pl
from jax.experimental.pallas import tpu as pltpu
from jax.experimental.pallas import tpu_sc as plsc

N_IN_ROWS = 50000
N_OUT_ROWS = 400000
FEAT = 128

NUM_CORES = 2
NUM_SUBCORES = 16
NUM_WORKERS = NUM_CORES * NUM_SUBCORES

CHUNK = 128
CHUNKS_PER_WORKER = 100
ROWS_PER_WORKER = CHUNK * CHUNKS_PER_WORKER
PADDED_ROWS = ROWS_PER_WORKER * NUM_WORKERS
NBUF = 4


def _gather_body(table_hbm, idx_hbm, out_hbm, idx_v, bufs, gat_sems, wb_sems, idx_sem):
    wid = lax.axis_index("s") * NUM_CORES + lax.axis_index("c")
    base = wid * ROWS_PER_WORKER

    pltpu.async_copy(idx_hbm.at[pl.ds(base, ROWS_PER_WORKER)], idx_v, idx_sem).wait()

    def start_gather(c, b):
        return pltpu.make_async_copy(
            table_hbm.at[idx_v.at[pl.ds(c * CHUNK, CHUNK)]], bufs[b], gat_sems[b]
        )

    def start_wb(c, b):
        return pltpu.make_async_copy(
            bufs[b], out_hbm.at[pl.ds(base + c * CHUNK, CHUNK)], wb_sems[b]
        )

    def ring_step(g, _):
        for b in range(NBUF):
            c = g * NBUF + b

            @pl.when(g >= 1)
            def _wait_buf_free():
                start_wb(0, b).wait()

            start_gather(c, b).start()

            prev_b = (b - 1) % NBUF
            if b == 0:

                @pl.when(g >= 1)
                def _process_prev():
                    start_gather(0, prev_b).wait()
                    start_wb((g - 1) * NBUF + NBUF - 1, prev_b).start()

            else:
                start_gather(0, prev_b).wait()
                start_wb(c - 1, prev_b).start()
        return 0

    lax.fori_loop(0, CHUNKS_PER_WORKER // NBUF, ring_step, 0)
    last = CHUNKS_PER_WORKER - 1
    start_gather(0, NBUF - 1).wait()
    start_wb(last, NBUF - 1).start()
    for b in range(NBUF):
        start_wb(0, b).wait()


@jax.jit
def _unpool_gather(table, idx):
    mesh = plsc.VectorSubcoreMesh(core_axis_name="c", subcore_axis_name="s")
    run = functools.partial(
        pl.kernel,
        mesh=mesh,
        out_type=jax.ShapeDtypeStruct((PADDED_ROWS, FEAT), jnp.float32),
        scratch_types=[
            pltpu.VMEM((ROWS_PER_WORKER,), jnp.int32),
            [pltpu.VMEM((CHUNK, FEAT), jnp.float32) for _ in range(NBUF)],
            [pltpu.SemaphoreType.DMA for _ in range(NBUF)],
            [pltpu.SemaphoreType.DMA for _ in range(NBUF)],
            pltpu.SemaphoreType.DMA,
        ],
    )(_gather_body)
    return run(table, idx)


def kernel(input_features, unpool_map):
    idx = unpool_map.astype(jnp.int32)
    pad = PADDED_ROWS - N_OUT_ROWS
    pad_idx = jnp.arange(pad, dtype=jnp.int32) % N_IN_ROWS
    idx_full = jnp.concatenate([idx, pad_idx])
    out = _unpool_gather(input_features, idx_full)
    return out[:N_OUT_ROWS]


# speedup vs baseline: 3.2069x; 1.0045x over previous
import functools

import jax
import jax.numpy as jnp
from jax import lax
from jax.experimental import pallas as pl
from jax.experimental.pallas import tpu as pltpu
from jax.experimental.pallas import tpu_sc as plsc

N_IN_ROWS = 50000
N_OUT_ROWS = 400000
FEAT = 128

NUM_CORES = 2
NUM_SUBCORES = 16
NUM_WORKERS = NUM_CORES * NUM_SUBCORES

CHUNK = 128
CHUNKS_PER_WORKER = 100
ROWS_PER_WORKER = CHUNK * CHUNKS_PER_WORKER
PADDED_ROWS = ROWS_PER_WORKER * NUM_WORKERS
NBUF = 5
DEPTH = 2


def _gather_body(table_hbm, idx_hbm, out_hbm, idx_v, bufs, gat_sems, wb_sems, idx_sem):
    wid = lax.axis_index("s") * NUM_CORES + lax.axis_index("c")
    base = wid * ROWS_PER_WORKER

    pltpu.async_copy(idx_hbm.at[pl.ds(base, ROWS_PER_WORKER)], idx_v, idx_sem).wait()

    def start_gather(c, b):
        return pltpu.make_async_copy(
            table_hbm.at[idx_v.at[pl.ds(c * CHUNK, CHUNK)]], bufs[b], gat_sems[b]
        )

    def start_wb(c, b):
        return pltpu.make_async_copy(
            bufs[b], out_hbm.at[pl.ds(base + c * CHUNK, CHUNK)], wb_sems[b]
        )

    def ring_step(g, _):
        for b in range(NBUF):
            c = g * NBUF + b

            @pl.when(g >= 1)
            def _wait_buf_free():
                start_wb(0, b).wait()

            start_gather(c, b).start()

            pb = (b - DEPTH) % NBUF
            p = g * NBUF + (b - DEPTH)

            def _process_prev(p=p, pb=pb):
                start_gather(0, pb).wait()
                start_wb(p, pb).start()

            if b >= DEPTH:
                _process_prev()
            else:
                pl.when(g >= 1)(_process_prev)
        return 0

    lax.fori_loop(0, CHUNKS_PER_WORKER // NBUF, ring_step, 0)
    for p in range(CHUNKS_PER_WORKER - DEPTH, CHUNKS_PER_WORKER):
        pb = p % NBUF
        start_gather(0, pb).wait()
        start_wb(p, pb).start()
    for b in range(NBUF):
        start_wb(0, b).wait()


@jax.jit
def _unpool_gather(table, idx):
    mesh = plsc.VectorSubcoreMesh(core_axis_name="c", subcore_axis_name="s")
    run = functools.partial(
        pl.kernel,
        mesh=mesh,
        out_type=jax.ShapeDtypeStruct((PADDED_ROWS, FEAT), jnp.float32),
        scratch_types=[
            pltpu.VMEM((ROWS_PER_WORKER,), jnp.int32),
            [pltpu.VMEM((CHUNK, FEAT), jnp.float32) for _ in range(NBUF)],
            [pltpu.SemaphoreType.DMA for _ in range(NBUF)],
            [pltpu.SemaphoreType.DMA for _ in range(NBUF)],
            pltpu.SemaphoreType.DMA,
        ],
    )(_gather_body)
    return run(table, idx)


def kernel(input_features, unpool_map):
    idx = unpool_map.astype(jnp.int32)
    pad = PADDED_ROWS - N_OUT_ROWS
    pad_idx = jnp.arange(pad, dtype=jnp.int32) % N_IN_ROWS
    idx_full = jnp.concatenate([idx, pad_idx])
    out = _unpool_gather(input_features, idx_full)
    return out[:N_OUT_ROWS]


# speedup vs baseline: 5.8209x; 1.8151x over previous
import functools

import jax
import jax.numpy as jnp
from jax import lax
from jax.experimental import pallas as pl
from jax.experimental.pallas import tpu as pltpu
from jax.experimental.pallas import tpu_sc as plsc

N_IN_ROWS = 50000
N_OUT_ROWS = 400000
FEAT = 128

NUM_CORES = 2
NUM_SUBCORES = 16
NUM_WORKERS = NUM_CORES * NUM_SUBCORES

CHUNK = 128
NUM_CHUNKS = N_OUT_ROWS // CHUNK
STEPS = 98
NBUF = 7
DEPTH = 2


def _gather_body(table_hbm, idx_hbm, out_hbm, idxbufs, rowbufs, isems, gsems, wsems):
    wid = lax.axis_index("s") * NUM_CORES + lax.axis_index("c")

    def chunk_row0(k):
        c = wid * STEPS + k
        c = jnp.where(c < NUM_CHUNKS, c, c - NUM_CHUNKS)
        return c * CHUNK

    def idx_copy(k, b):
        return pltpu.make_async_copy(
            idx_hbm.at[pl.ds(chunk_row0(k), CHUNK)], idxbufs[b], isems[b]
        )

    def gather_copy(k, b):
        return pltpu.make_async_copy(
            table_hbm.at[idxbufs[b]], rowbufs[b], gsems[b]
        )

    def wb_copy(k, b):
        return pltpu.make_async_copy(
            rowbufs[b], out_hbm.at[pl.ds(chunk_row0(k), CHUNK)], wsems[b]
        )

    def step(k_static_b, g):
        b = k_static_b
        k = g * NBUF + b

        @pl.when(g >= 1)
        def _wait_buf_free():
            wb_copy(0, b).wait()

        idx_copy(k, b).start()

        b1 = (b - 1) % NBUF

        def _launch_gather():
            idx_copy(0, b1).wait()
            gather_copy(0, b1).start()

        if b >= 1:
            _launch_gather()
        else:
            pl.when(g >= 1)(_launch_gather)

        b2 = (b - 1 - DEPTH) % NBUF
        p = k - 1 - DEPTH

        def _writeback():
            gather_copy(0, b2).wait()
            wb_copy(p, b2).start()

        if b >= 1 + DEPTH:
            _writeback()
        else:
            pl.when(g >= 1)(_writeback)

    def ring_pass(g, _):
        for b in range(NBUF):
            step(b, g)
        return 0

    lax.fori_loop(0, STEPS // NBUF, ring_pass, 0)

    bl = (STEPS - 1) % NBUF
    idx_copy(0, bl).wait()
    gather_copy(0, bl).start()
    for p in range(STEPS - 1 - DEPTH, STEPS):
        pb = p % NBUF
        gather_copy(0, pb).wait()
        wb_copy(p, pb).start()
    for b in range(NBUF):
        wb_copy(0, b).wait()


@jax.jit
def _unpool_gather(table, idx):
    mesh = plsc.VectorSubcoreMesh(core_axis_name="c", subcore_axis_name="s")
    run = functools.partial(
        pl.kernel,
        mesh=mesh,
        out_type=jax.ShapeDtypeStruct((N_OUT_ROWS, FEAT), jnp.float32),
        scratch_types=[
            [pltpu.VMEM((CHUNK,), jnp.int32) for _ in range(NBUF)],
            [pltpu.VMEM((CHUNK, FEAT), jnp.float32) for _ in range(NBUF)],
            [pltpu.SemaphoreType.DMA for _ in range(NBUF)],
            [pltpu.SemaphoreType.DMA for _ in range(NBUF)],
            [pltpu.SemaphoreType.DMA for _ in range(NBUF)],
        ],
    )(_gather_body)
    return run(table, idx)


def kernel(input_features, unpool_map):
    return _unpool_gather(input_features, unpool_map.astype(jnp.int32))
